# baseline (speedup 1.0000x reference)
import jax
import jax.numpy as jnp
from jax.experimental import pallas as pl
from jax.experimental.pallas import tpu as pltpu


def _body(pred_ref, lab_ref, n_obj_ref, n_noobj_ref, s_obj_ref, s_noobj_ref,
          s_bbox_ref, s_cls_ref):
    i = pl.program_id(0)

    @pl.when(i == 0)
    def _init():
        for r in (n_obj_ref, n_noobj_ref, s_obj_ref, s_noobj_ref,
                  s_bbox_ref, s_cls_ref):
            r[0, 0] = 0.0

    p = pred_ref[0]
    l = lab_ref[0]

    t = l[:, 4]
    p4 = p[:, 4]
    obj = (t == 1.0).astype(jnp.float32)
    noobj = (t == 10.0).astype(jnp.float32)

    l1 = jnp.maximum(jnp.log(p4), -100.0)
    l2 = jnp.maximum(jnp.log(1.0 - p4), -100.0)
    bce = -(t * l1 + (1.0 - t) * l2)

    sq = (jnp.sqrt(p[:, 0:4]) - jnp.sqrt(l[:, 0:4])) ** 2

    logits = p[:, 5:85]
    lcls = l[:, 5:85]
    lse = jnp.log(jnp.sum(jnp.exp(logits), axis=-1))
    mx = jnp.max(lcls, axis=-1, keepdims=True)
    lanes = jax.lax.broadcasted_iota(jnp.int32, lcls.shape, 1)
    idx = jnp.min(jnp.where(lcls == mx, lanes, 80), axis=-1, keepdims=True)
    picked = jnp.sum(jnp.where(lanes == idx, logits, 0.0), axis=-1)

    n_obj_ref[0, 0] += jnp.sum(obj)
    n_noobj_ref[0, 0] += jnp.sum(noobj)
    s_obj_ref[0, 0] += jnp.sum(bce * obj)
    s_noobj_ref[0, 0] += jnp.sum(bce * noobj)
    s_bbox_ref[0, 0] += jnp.sum(sq * obj[:, None])
    s_cls_ref[0, 0] += jnp.sum((lse - picked) * obj)


def kernel(prediction, label):
    B, A, C = prediction.shape
    scalar = jax.ShapeDtypeStruct((1, 1), jnp.float32)
    outs = pl.pallas_call(
        _body,
        grid=(B,),
        in_specs=[
            pl.BlockSpec((1, A, C), lambda i: (i, 0, 0)),
            pl.BlockSpec((1, A, C), lambda i: (i, 0, 0)),
        ],
        out_specs=[pl.BlockSpec((1, 1), lambda i: (0, 0),
                                memory_space=pltpu.SMEM)] * 6,
        out_shape=[scalar] * 6,
    )(prediction, label)
    n_obj, n_noobj, s_obj, s_noobj, s_bbox, s_cls = [o[0, 0] for o in outs]
    return (5.0 * s_bbox / (4.0 * n_obj) + s_obj / n_obj
            + 5.0 * s_noobj / n_noobj + s_cls / n_obj)
